# Initial kernel scaffold; baseline (speedup 1.0000x reference)
import jax
import jax.numpy as jnp
from jax.experimental import pallas as pl


def kernel(x, neighbors, Wih0, Whh0, bih0, bhh0, Wl0, bl0, Wih1, Whh1, bih1, bhh1, Wl1, bl1, Wih2, Whh2, bih2, bhh2, Wl2, bl2, Wout, bout):
    raise NotImplementedError("write your pallas kernel here")


# speedup vs baseline: 5.0928x; 5.0928x over previous
import functools

import jax
import jax.numpy as jnp
from jax import lax
from jax.experimental import pallas as pl
from jax.experimental.pallas import tpu as pltpu
from jax.experimental.pallas import tpu_sc as plsc

N = 10000
DEG = 16
H = 128
G4 = 4 * H

CH = 80
NCHUNKS = N // CH
NW = 32
KMAX = (NCHUNKS + NW - 1) // NW


def _sc_gather_body(nbr_hbm, h_hbm, seq_hbm, nbr_v, idx2d, buf0, buf1,
                    sem0, sem1):
    wid = lax.axis_index("s") * 2 + lax.axis_index("c")
    iota16 = lax.iota(jnp.int32, 16)
    bufs = (buf0, buf1)
    sems = (sem0, sem1)

    for k in range(KMAX):
        ci = wid + k * NW

        @pl.when(ci < NCHUNKS)
        def _chunk():
            base = ci * CH
            pltpu.sync_copy(nbr_hbm.at[pl.ds(base, CH)], nbr_v)

            def _node(j, carry):
                row = nbr_v[j]
                srt, _ = plsc.sort_key_val(row, row)
                plsc.store_scatter(
                    idx2d, [iota16, jnp.full((16,), j, jnp.int32)], srt)
                return carry

            lax.fori_loop(0, CH, _node, 0)

            cps = [None, None]
            for t in range(DEG):
                b = t % 2
                if cps[b] is not None:
                    cps[b].wait()
                    pltpu.sync_copy(bufs[b],
                                    seq_hbm.at[t - 2, pl.ds(base, CH)])
                cps[b] = pltpu.async_copy(h_hbm.at[idx2d.at[t]], bufs[b],
                                          sems[b])
            for t in (DEG - 2, DEG - 1):
                b = t % 2
                cps[b].wait()
                pltpu.sync_copy(bufs[b], seq_hbm.at[t, pl.ds(base, CH)])


@functools.cache
def _sc_gather_kernel():
    return pl.kernel(
        _sc_gather_body,
        mesh=plsc.VectorSubcoreMesh(core_axis_name="c", subcore_axis_name="s"),
        compiler_params=pltpu.CompilerParams(needs_layout_passes=False),
        out_type=jax.ShapeDtypeStruct((DEG, N, H), jnp.float32),
        scratch_types=[
            pltpu.VMEM((CH, DEG), jnp.int32),
            pltpu.VMEM((DEG, CH), jnp.int32),
            pltpu.VMEM((CH, H), jnp.float32),
            pltpu.VMEM((CH, H), jnp.float32),
            pltpu.SemaphoreType.DMA,
            pltpu.SemaphoreType.DMA,
        ],
    )


def _sc_gather(neighbors, h):
    return _sc_gather_kernel()(neighbors, h)


BN = 1000


def _lstm_steps(seq_ref, wcat_ref, bias_ref):
    hs = jnp.zeros((BN, H), jnp.float32)
    cs = jnp.zeros((BN, H), jnp.float32)
    for t in range(DEG):
        xt = seq_ref[t]
        xh = jnp.concatenate([xt, hs], axis=1)
        g = jnp.dot(xh, wcat_ref[...],
                    preferred_element_type=jnp.float32) + bias_ref[...]
        i_g = g[:, :H]
        f_g = g[:, H:2 * H]
        g_g = g[:, 2 * H:3 * H]
        o_g = g[:, 3 * H:]
        cs = jax.nn.sigmoid(f_g) * cs + jax.nn.sigmoid(i_g) * jnp.tanh(g_g)
        hs = jax.nn.sigmoid(o_g) * jnp.tanh(cs)
    return hs


def _layer_body(seq_ref, h_ref, wcat_ref, bias_ref, wl_ref, bl_ref, out_ref):
    hs = _lstm_steps(seq_ref, wcat_ref, bias_ref)
    xh2 = jnp.concatenate([h_ref[...], hs], axis=1)
    out = jnp.dot(xh2, wl_ref[...],
                  preferred_element_type=jnp.float32) + bl_ref[...]
    out_ref[...] = jnp.maximum(out, 0.0)


def _final_body(seq_ref, h_ref, wcat_ref, bias_ref, wl_ref, bl_ref,
                wout_ref, bout_ref, out_ref):
    hs = _lstm_steps(seq_ref, wcat_ref, bias_ref)
    xh2 = jnp.concatenate([h_ref[...], hs], axis=1)
    hnew = jnp.maximum(
        jnp.dot(xh2, wl_ref[...],
                preferred_element_type=jnp.float32) + bl_ref[...], 0.0)
    out_ref[...] = jnp.dot(hnew, wout_ref[...],
                           preferred_element_type=jnp.float32) + bout_ref[...]


def _common_specs():
    return [
        pl.BlockSpec((DEG, BN, H), lambda i: (0, i, 0)),
        pl.BlockSpec((BN, H), lambda i: (i, 0)),
        pl.BlockSpec((2 * H, G4), lambda i: (0, 0)),
        pl.BlockSpec((1, G4), lambda i: (0, 0)),
        pl.BlockSpec((2 * H, H), lambda i: (0, 0)),
        pl.BlockSpec((1, H), lambda i: (0, 0)),
    ]


def _layer_call(seq, h, wcat, bias, wl_t, bl_r):
    return pl.pallas_call(
        _layer_body,
        grid=(N // BN,),
        in_specs=_common_specs(),
        out_specs=pl.BlockSpec((BN, H), lambda i: (i, 0)),
        out_shape=jax.ShapeDtypeStruct((N, H), jnp.float32),
    )(seq, h, wcat, bias, wl_t, bl_r)


def _final_call(seq, h, wcat, bias, wl_t, bl_r, wout_p, bout_p):
    return pl.pallas_call(
        _final_body,
        grid=(N // BN,),
        in_specs=_common_specs() + [
            pl.BlockSpec((H, H), lambda i: (0, 0)),
            pl.BlockSpec((1, H), lambda i: (0, 0)),
        ],
        out_specs=pl.BlockSpec((BN, H), lambda i: (i, 0)),
        out_shape=jax.ShapeDtypeStruct((N, H), jnp.float32),
    )(seq, h, wcat, bias, wl_t, bl_r, wout_p, bout_p)


def kernel(x, neighbors, Wih0, Whh0, bih0, bhh0, Wl0, bl0,
           Wih1, Whh1, bih1, bhh1, Wl1, bl1,
           Wih2, Whh2, bih2, bhh2, Wl2, bl2, Wout, bout):
    params = [
        (Wih0, Whh0, bih0, bhh0, Wl0, bl0),
        (Wih1, Whh1, bih1, bhh1, Wl1, bl1),
        (Wih2, Whh2, bih2, bhh2, Wl2, bl2),
    ]
    wout_p = jnp.zeros((H, H), jnp.float32).at[:, 0].set(Wout[0])
    bout_p = jnp.zeros((1, H), jnp.float32).at[0, 0].set(bout[0])

    h = x
    for l, (Wih, Whh, bih, bhh, Wl, bl) in enumerate(params):
        wcat = jnp.concatenate([Wih.T, Whh.T], axis=0)
        bias = (bih + bhh).reshape(1, G4)
        wl_t = Wl.T
        bl_r = bl.reshape(1, H)
        seq = _sc_gather(neighbors, h)
        if l < 2:
            h = _layer_call(seq, h, wcat, bias, wl_t, bl_r)
        else:
            out = _final_call(seq, h, wcat, bias, wl_t, bl_r, wout_p, bout_p)
    return out[:, 0]


# speedup vs baseline: 6.0643x; 1.1908x over previous
import functools

import jax
import jax.numpy as jnp
from jax import lax
from jax.experimental import pallas as pl
from jax.experimental.pallas import tpu as pltpu
from jax.experimental.pallas import tpu_sc as plsc

N = 10000
DEG = 16
H = 128
G4 = 4 * H

CH = 80
NCHUNKS = N // CH
NW = 32
KMAX = (NCHUNKS + NW - 1) // NW


def _sc_gather_body(nbr_hbm, h_hbm, seq_hbm, nbr_v, idxall, buf0, buf1,
                    shared, gsem0, gsem1, ssem0, ssem1):
    sid = lax.axis_index("s")
    wid = sid * 2 + lax.axis_index("c")
    iota16 = lax.iota(jnp.int32, 16)
    bufs = (buf0, buf1)
    gsems = (gsem0, gsem1)
    ssems = (ssem0, ssem1)

    @pl.when(sid < 15)
    def _stage():
        pltpu.sync_copy(h_hbm.at[pl.ds(sid * 624, 624)],
                        shared.at[pl.ds(sid * 624, 624)])

    @pl.when(sid == 15)
    def _stage_last():
        pltpu.sync_copy(h_hbm.at[pl.ds(9360, 640)],
                        shared.at[pl.ds(9360, 640)])

    for k in range(KMAX):
        ci = wid + k * NW

        @pl.when(ci < NCHUNKS)
        def _sort_chunk():
            pltpu.sync_copy(nbr_hbm.at[pl.ds(ci * CH, CH)], nbr_v)

            def _node(j, carry):
                row = nbr_v[j]
                srt, _ = plsc.sort_key_val(row, row)
                plsc.store_scatter(
                    idxall, [k * DEG + iota16, jnp.full((16,), j, jnp.int32)],
                    srt)
                return carry

            lax.fori_loop(0, CH, _node, 0)

    plsc.subcore_barrier()

    for k in range(KMAX):
        ci = wid + k * NW

        @pl.when(ci < NCHUNKS)
        def _chunk():
            base = ci * CH
            gath = [None, None]
            scat = [None, None]
            for t in range(DEG):
                b = t % 2
                if scat[b] is not None:
                    scat[b].wait()
                gath[b] = pltpu.async_copy(
                    shared.at[idxall.at[k * DEG + t]], bufs[b], gsems[b])
                o = 1 - b
                if gath[o] is not None:
                    gath[o].wait()
                    scat[o] = pltpu.async_copy(
                        bufs[o], seq_hbm.at[t - 1, pl.ds(base, CH)], ssems[o])
            b = (DEG - 1) % 2
            gath[b].wait()
            scat[b] = pltpu.async_copy(
                bufs[b], seq_hbm.at[DEG - 1, pl.ds(base, CH)], ssems[b])
            scat[0].wait()
            scat[1].wait()


@functools.cache
def _sc_gather_kernel():
    return pl.kernel(
        _sc_gather_body,
        mesh=plsc.VectorSubcoreMesh(core_axis_name="c", subcore_axis_name="s"),
        compiler_params=pltpu.CompilerParams(needs_layout_passes=False),
        out_type=jax.ShapeDtypeStruct((DEG, N, H), jnp.float32),
        scratch_types=[
            pltpu.VMEM((CH, DEG), jnp.int32),
            pltpu.VMEM((KMAX * DEG, CH), jnp.int32),
            pltpu.VMEM((CH, H), jnp.float32),
            pltpu.VMEM((CH, H), jnp.float32),
            pltpu.VMEM_SHARED((N, H), jnp.float32),
            pltpu.SemaphoreType.DMA,
            pltpu.SemaphoreType.DMA,
            pltpu.SemaphoreType.DMA,
            pltpu.SemaphoreType.DMA,
        ],
    )


def _sc_gather(neighbors, h):
    return _sc_gather_kernel()(neighbors, h)


BN = 1000


def _lstm_steps(seq_ref, wcat_ref, bias_ref):
    hs = jnp.zeros((BN, H), jnp.float32)
    cs = jnp.zeros((BN, H), jnp.float32)
    for t in range(DEG):
        xt = seq_ref[t]
        xh = jnp.concatenate([xt, hs], axis=1)
        g = jnp.dot(xh, wcat_ref[...],
                    preferred_element_type=jnp.float32) + bias_ref[...]
        i_g = g[:, :H]
        f_g = g[:, H:2 * H]
        g_g = g[:, 2 * H:3 * H]
        o_g = g[:, 3 * H:]
        cs = jax.nn.sigmoid(f_g) * cs + jax.nn.sigmoid(i_g) * jnp.tanh(g_g)
        hs = jax.nn.sigmoid(o_g) * jnp.tanh(cs)
    return hs


def _layer_body(seq_ref, h_ref, wcat_ref, bias_ref, wl_ref, bl_ref, out_ref):
    hs = _lstm_steps(seq_ref, wcat_ref, bias_ref)
    xh2 = jnp.concatenate([h_ref[...], hs], axis=1)
    out = jnp.dot(xh2, wl_ref[...],
                  preferred_element_type=jnp.float32) + bl_ref[...]
    out_ref[...] = jnp.maximum(out, 0.0)


def _final_body(seq_ref, h_ref, wcat_ref, bias_ref, wl_ref, bl_ref,
                wout_ref, bout_ref, out_ref):
    hs = _lstm_steps(seq_ref, wcat_ref, bias_ref)
    xh2 = jnp.concatenate([h_ref[...], hs], axis=1)
    hnew = jnp.maximum(
        jnp.dot(xh2, wl_ref[...],
                preferred_element_type=jnp.float32) + bl_ref[...], 0.0)
    out_ref[...] = jnp.dot(hnew, wout_ref[...],
                           preferred_element_type=jnp.float32) + bout_ref[...]


def _common_specs():
    return [
        pl.BlockSpec((DEG, BN, H), lambda i: (0, i, 0)),
        pl.BlockSpec((BN, H), lambda i: (i, 0)),
        pl.BlockSpec((2 * H, G4), lambda i: (0, 0)),
        pl.BlockSpec((1, G4), lambda i: (0, 0)),
        pl.BlockSpec((2 * H, H), lambda i: (0, 0)),
        pl.BlockSpec((1, H), lambda i: (0, 0)),
    ]


def _layer_call(seq, h, wcat, bias, wl_t, bl_r):
    return pl.pallas_call(
        _layer_body,
        grid=(N // BN,),
        in_specs=_common_specs(),
        out_specs=pl.BlockSpec((BN, H), lambda i: (i, 0)),
        out_shape=jax.ShapeDtypeStruct((N, H), jnp.float32),
    )(seq, h, wcat, bias, wl_t, bl_r)


def _final_call(seq, h, wcat, bias, wl_t, bl_r, wout_p, bout_p):
    return pl.pallas_call(
        _final_body,
        grid=(N // BN,),
        in_specs=_common_specs() + [
            pl.BlockSpec((H, H), lambda i: (0, 0)),
            pl.BlockSpec((1, H), lambda i: (0, 0)),
        ],
        out_specs=pl.BlockSpec((BN, H), lambda i: (i, 0)),
        out_shape=jax.ShapeDtypeStruct((N, H), jnp.float32),
    )(seq, h, wcat, bias, wl_t, bl_r, wout_p, bout_p)


def kernel(x, neighbors, Wih0, Whh0, bih0, bhh0, Wl0, bl0,
           Wih1, Whh1, bih1, bhh1, Wl1, bl1,
           Wih2, Whh2, bih2, bhh2, Wl2, bl2, Wout, bout):
    params = [
        (Wih0, Whh0, bih0, bhh0, Wl0, bl0),
        (Wih1, Whh1, bih1, bhh1, Wl1, bl1),
        (Wih2, Whh2, bih2, bhh2, Wl2, bl2),
    ]
    wout_p = jnp.zeros((H, H), jnp.float32).at[:, 0].set(Wout[0])
    bout_p = jnp.zeros((1, H), jnp.float32).at[0, 0].set(bout[0])

    h = x
    for l, (Wih, Whh, bih, bhh, Wl, bl) in enumerate(params):
        wcat = jnp.concatenate([Wih.T, Whh.T], axis=0)
        bias = (bih + bhh).reshape(1, G4)
        wl_t = Wl.T
        bl_r = bl.reshape(1, H)
        seq = _sc_gather(neighbors, h)
        if l < 2:
            h = _layer_call(seq, h, wcat, bias, wl_t, bl_r)
        else:
            out = _final_call(seq, h, wcat, bias, wl_t, bl_r, wout_p, bout_p)
    return out[:, 0]


# speedup vs baseline: 7.5360x; 1.2427x over previous
import functools

import jax
import jax.numpy as jnp
from jax import lax
from jax.experimental import pallas as pl
from jax.experimental.pallas import tpu as pltpu
from jax.experimental.pallas import tpu_sc as plsc

N = 10000
DEG = 16
H = 128
G4 = 4 * H

CH = 80
NCHUNKS = N // CH
NW = 32
KMAX = (NCHUNKS + NW - 1) // NW


def _sc_gather_body(nbr_hbm, h_hbm, seq_hbm, nbr_v, idxall, buf0, buf1,
                    shared, gsem0, gsem1, ssem0, ssem1):
    sid = lax.axis_index("s")
    wid = sid * 2 + lax.axis_index("c")
    iota16 = lax.iota(jnp.int32, 16)
    bufs = (buf0, buf1)
    gsems = (gsem0, gsem1)
    ssems = (ssem0, ssem1)

    @pl.when(sid < 15)
    def _stage():
        pltpu.sync_copy(h_hbm.at[pl.ds(sid * 624, 624)],
                        shared.at[pl.ds(sid * 624, 624)])

    @pl.when(sid == 15)
    def _stage_last():
        pltpu.sync_copy(h_hbm.at[pl.ds(9360, 640)],
                        shared.at[pl.ds(9360, 640)])

    for k in range(KMAX):
        ci = wid + k * NW

        @pl.when(ci < NCHUNKS)
        def _sort_chunk():
            pltpu.sync_copy(nbr_hbm.at[pl.ds(ci * CH, CH)], nbr_v)

            def _node(j, carry):
                row = nbr_v[j]
                srt, _ = plsc.sort_key_val(row, row)
                plsc.store_scatter(
                    idxall, [k * DEG + iota16, jnp.full((16,), j, jnp.int32)],
                    srt)
                return carry

            lax.fori_loop(0, CH, _node, 0)

    plsc.subcore_barrier()

    for k in range(KMAX):
        ci = wid + k * NW

        @pl.when(ci < NCHUNKS)
        def _chunk():
            base = ci * CH
            gath = [None, None]
            scat = [None, None]
            for t in range(DEG):
                b = t % 2
                if scat[b] is not None:
                    scat[b].wait()
                gath[b] = pltpu.async_copy(
                    shared.at[idxall.at[k * DEG + t]], bufs[b], gsems[b])
                o = 1 - b
                if gath[o] is not None:
                    gath[o].wait()
                    scat[o] = pltpu.async_copy(
                        bufs[o], seq_hbm.at[t - 1, pl.ds(base, CH)], ssems[o])
            b = (DEG - 1) % 2
            gath[b].wait()
            scat[b] = pltpu.async_copy(
                bufs[b], seq_hbm.at[DEG - 1, pl.ds(base, CH)], ssems[b])
            scat[0].wait()
            scat[1].wait()


@functools.cache
def _sc_gather_kernel():
    return pl.kernel(
        _sc_gather_body,
        mesh=plsc.VectorSubcoreMesh(core_axis_name="c", subcore_axis_name="s"),
        compiler_params=pltpu.CompilerParams(needs_layout_passes=False),
        out_type=jax.ShapeDtypeStruct((DEG, N, H), jnp.float32),
        scratch_types=[
            pltpu.VMEM((CH, DEG), jnp.int32),
            pltpu.VMEM((KMAX * DEG, CH), jnp.int32),
            pltpu.VMEM((CH, H), jnp.float32),
            pltpu.VMEM((CH, H), jnp.float32),
            pltpu.VMEM_SHARED((N, H), jnp.float32),
            pltpu.SemaphoreType.DMA,
            pltpu.SemaphoreType.DMA,
            pltpu.SemaphoreType.DMA,
            pltpu.SemaphoreType.DMA,
        ],
    )


def _sc_gather(neighbors, h):
    return _sc_gather_kernel()(neighbors, h)


BN = 1000


def _lstm_steps(seq_ref, wcat_ref, bias_ref):
    HS = jnp.zeros((BN, H), jnp.float32)
    cs = jnp.zeros((BN, H), jnp.float32)
    for t in range(DEG):
        xt = seq_ref[t]
        xh = jnp.concatenate([xt, HS], axis=1)
        g = jnp.dot(xh, wcat_ref[...],
                    preferred_element_type=jnp.float32) + bias_ref[...]
        ti = jnp.tanh(g[:, :H])
        tf = jnp.tanh(g[:, H:2 * H])
        tg = jnp.tanh(g[:, 2 * H:3 * H])
        to = jnp.tanh(g[:, 3 * H:])
        cs = 0.5 * ((tf + 1.0) * cs + (ti + 1.0) * tg)
        HS = (to + 1.0) * jnp.tanh(cs)
    return HS


def _layer_body(seq_ref, h_ref, wcat_ref, bias_ref, wl_ref, bl_ref, out_ref):
    hs = _lstm_steps(seq_ref, wcat_ref, bias_ref)
    xh2 = jnp.concatenate([h_ref[...], hs], axis=1)
    out = jnp.dot(xh2, wl_ref[...],
                  preferred_element_type=jnp.float32) + bl_ref[...]
    out_ref[...] = jnp.maximum(out, 0.0)


def _final_body(seq_ref, h_ref, wcat_ref, bias_ref, wl_ref, bl_ref,
                wout_ref, bout_ref, out_ref):
    hs = _lstm_steps(seq_ref, wcat_ref, bias_ref)
    xh2 = jnp.concatenate([h_ref[...], hs], axis=1)
    hnew = jnp.maximum(
        jnp.dot(xh2, wl_ref[...],
                preferred_element_type=jnp.float32) + bl_ref[...], 0.0)
    out_ref[...] = jnp.dot(hnew, wout_ref[...],
                           preferred_element_type=jnp.float32) + bout_ref[...]


def _common_specs():
    return [
        pl.BlockSpec((DEG, BN, H), lambda i: (0, i, 0)),
        pl.BlockSpec((BN, H), lambda i: (i, 0)),
        pl.BlockSpec((2 * H, G4), lambda i: (0, 0)),
        pl.BlockSpec((1, G4), lambda i: (0, 0)),
        pl.BlockSpec((2 * H, H), lambda i: (0, 0)),
        pl.BlockSpec((1, H), lambda i: (0, 0)),
    ]


def _layer_call(seq, h, wcat, bias, wl_t, bl_r):
    return pl.pallas_call(
        _layer_body,
        grid=(N // BN,),
        in_specs=_common_specs(),
        out_specs=pl.BlockSpec((BN, H), lambda i: (i, 0)),
        out_shape=jax.ShapeDtypeStruct((N, H), jnp.float32),
    )(seq, h, wcat, bias, wl_t, bl_r)


def _final_call(seq, h, wcat, bias, wl_t, bl_r, wout_p, bout_p):
    return pl.pallas_call(
        _final_body,
        grid=(N // BN,),
        in_specs=_common_specs() + [
            pl.BlockSpec((H, H), lambda i: (0, 0)),
            pl.BlockSpec((1, H), lambda i: (0, 0)),
        ],
        out_specs=pl.BlockSpec((BN, H), lambda i: (i, 0)),
        out_shape=jax.ShapeDtypeStruct((N, H), jnp.float32),
    )(seq, h, wcat, bias, wl_t, bl_r, wout_p, bout_p)


def kernel(x, neighbors, Wih0, Whh0, bih0, bhh0, Wl0, bl0,
           Wih1, Whh1, bih1, bhh1, Wl1, bl1,
           Wih2, Whh2, bih2, bhh2, Wl2, bl2, Wout, bout):
    params = [
        (Wih0, Whh0, bih0, bhh0, Wl0, bl0),
        (Wih1, Whh1, bih1, bhh1, Wl1, bl1),
        (Wih2, Whh2, bih2, bhh2, Wl2, bl2),
    ]
    wout_p = jnp.zeros((H, H), jnp.float32).at[:, 0].set(Wout[0])
    bout_p = jnp.zeros((1, H), jnp.float32).at[0, 0].set(bout[0])

    col_scale = jnp.concatenate([
        jnp.full((H,), 0.5, jnp.float32),
        jnp.full((H,), 0.5, jnp.float32),
        jnp.ones((H,), jnp.float32),
        jnp.full((H,), 0.5, jnp.float32),
    ])
    h = x
    for l, (Wih, Whh, bih, bhh, Wl, bl) in enumerate(params):
        wcat = (jnp.concatenate([Wih.T, 0.5 * Whh.T], axis=0)
                * col_scale[None, :])
        bias = ((bih + bhh) * col_scale).reshape(1, G4)
        wl_t = jnp.concatenate([Wl.T[:H], 0.5 * Wl.T[H:]], axis=0)
        bl_r = bl.reshape(1, H)
        seq = _sc_gather(neighbors, h)
        if l < 2:
            h = _layer_call(seq, h, wcat, bias, wl_t, bl_r)
        else:
            out = _final_call(seq, h, wcat, bias, wl_t, bl_r, wout_p, bout_p)
    return out[:, 0]
